# Initial kernel scaffold; baseline (speedup 1.0000x reference)
import jax
import jax.numpy as jnp
from jax.experimental import pallas as pl


def kernel(atomic_embedding, f_ij, pair_indices, W_attn, W_i, b_i, W_j, b_j, Wr1, br1, Wr2, br2, gate, W_v, b_v, Wo1, bo1, Wo2, bo2, W_out, b_out):
    raise NotImplementedError("write your pallas kernel here")


# speedup vs baseline: 2.7615x; 2.7615x over previous
import functools

import jax
import jax.numpy as jnp
from jax import lax
from jax.experimental import pallas as pl
from jax.experimental.pallas import tpu as pltpu
from jax.experimental.pallas import tpu_sc as plsc

_NC = 2
_NS = 16
_LL = 16

_C = 80
_BN = 1000
_BE = 8000


def _softplus(x):
    return jnp.maximum(x, 0.0) + jnp.log1p(jnp.exp(-jnp.abs(x)))


def _node_body(a_ref, wi_ref, bi_ref, wj_ref, bj_ref, emb_ref, xi_ref, tj_ref):
    emb = _softplus(a_ref[...])
    emb_ref[...] = emb
    xi_ref[...] = _softplus(
        jnp.dot(emb, wi_ref[...], preferred_element_type=jnp.float32) + bi_ref[...])
    tj_ref[...] = _softplus(
        jnp.dot(emb, wj_ref[...], preferred_element_type=jnp.float32) + bj_ref[...])


def _node_stage(a, W_i, b_i, W_j, b_j):
    n, d = a.shape
    row = pl.BlockSpec((_BN, d), lambda i: (i, 0))
    w = pl.BlockSpec((d, d), lambda i: (0, 0))
    b = pl.BlockSpec((1, d), lambda i: (0, 0))
    return pl.pallas_call(
        _node_body,
        grid=(n // _BN,),
        in_specs=[row, w, b, w, b],
        out_specs=[row, row, row],
        out_shape=[jax.ShapeDtypeStruct((n, d), jnp.float32)] * 3,
    )(a, W_i, b_i.reshape(1, d), W_j, b_j.reshape(1, d))


def _attn_body(f_ref, wa_ref, g_ref):
    g_ref[...] = jnp.dot(f_ref[...], wa_ref[...], preferred_element_type=jnp.float32)


def _attn_stage(f_ij, W_attn):
    e, r = f_ij.shape
    d = W_attn.shape[1]
    return pl.pallas_call(
        _attn_body,
        grid=(e // _BE,),
        in_specs=[pl.BlockSpec((_BE, r), lambda i: (i, 0)),
                  pl.BlockSpec((r, d), lambda i: (0, 0))],
        out_specs=pl.BlockSpec((_BE, d), lambda i: (i, 0)),
        out_shape=jax.ShapeDtypeStruct((e, d), jnp.float32),
    )(f_ij, W_attn)


def _edge_stage(t_j, g, idx_i3, idx_j3):
    n, d = t_j.shape
    e = g.shape[0]
    ept = e // (_NC * _NS)
    nchunk = ept // _C
    spt = -(-(-(-n // _NS)) // _C) * _C
    npad = spt * _NS
    nidx = 64
    mesh = plsc.VectorSubcoreMesh(core_axis_name="c", subcore_axis_name="s")

    @functools.partial(
        pl.kernel,
        mesh=mesh,
        out_type=jax.ShapeDtypeStruct((_NC, npad, d), jnp.float32),
        scratch_types=[
            pltpu.VMEM((nidx, _C), jnp.int32),
            pltpu.VMEM((nidx, _C), jnp.int32),
            pltpu.VMEM((_C, d), jnp.float32),
            pltpu.VMEM((_C, d), jnp.float32),
            pltpu.VMEM_SHARED((npad, d), jnp.float32),
            pltpu.SemaphoreType.DMA,
        ],
    )
    def edge_kernel(t_hbm, g_hbm, ii_hbm, jj_hbm, out_hbm,
                    ii_v, jj_v, rows_v, g_v, acc_sh, sem):
        c = lax.axis_index("c")
        s = lax.axis_index("s")
        wid = c * _NS + s
        soff = pl.multiple_of(s * spt, 8)

        def zfill(i, carry):
            for k in range(d // _LL):
                rows_v[i, pl.ds(k * _LL, _LL)] = jnp.zeros((_LL,), jnp.float32)
            return carry
        lax.fori_loop(0, _C, zfill, 0)
        for rr in range(spt // _C):
            pltpu.sync_copy(rows_v, acc_sh.at[pl.ds(soff + rr * _C, _C)])
        plsc.subcore_barrier()

        for goff in range(0, nchunk, nidx):
            cnt = min(nidx, nchunk - goff)
            pltpu.sync_copy(ii_hbm.at[wid, pl.ds(goff, cnt)],
                            ii_v.at[pl.ds(0, cnt)])
            pltpu.sync_copy(jj_hbm.at[wid, pl.ds(goff, cnt)],
                            jj_v.at[pl.ds(0, cnt)])

            def chunk(kk, carry):
                base = pl.multiple_of(wid * ept + (goff + kk) * _C, 8)
                pltpu.async_copy(t_hbm.at[jj_v.at[kk]], rows_v, sem).wait()
                pltpu.sync_copy(g_hbm.at[pl.ds(base, _C)], g_v)

                def mul(i, cc):
                    for k in range(d // _LL):
                        sl = pl.ds(k * _LL, _LL)
                        rows_v[i, sl] = rows_v[i, sl] * g_v[i, sl]
                    return cc
                lax.fori_loop(0, _C, mul, 0)
                pltpu.sync_copy(rows_v, acc_sh.at[ii_v.at[kk]], add=True)
                return carry
            lax.fori_loop(0, cnt, chunk, 0)

        plsc.subcore_barrier()
        pltpu.sync_copy(acc_sh.at[pl.ds(soff, spt)],
                        out_hbm.at[c, pl.ds(soff, spt)])

    return edge_kernel(t_j, g, idx_i3, idx_j3)


def _post_body(xi_ref, p_ref, emb_ref, wr1_ref, br1_ref, wr2_ref, br2_ref,
               gate_ref, wv_ref, bv_ref, wo1_ref, bo1_ref, wo2_ref, bo2_ref,
               wout_ref, bout_ref, ne_ref, pred_ref):
    upd = xi_ref[...] + p_ref[0] + p_ref[1]
    for k in range(wr1_ref.shape[0]):
        h = _softplus(upd)
        h = _softplus(
            jnp.dot(h, wr1_ref[k], preferred_element_type=jnp.float32) + br1_ref[k])
        h = jnp.dot(h, wr2_ref[k], preferred_element_type=jnp.float32) + br2_ref[k]
        upd = upd + h
    upd = _softplus(upd)
    ne = (gate_ref[...] * emb_ref[...]
          + jnp.dot(upd, wv_ref[...], preferred_element_type=jnp.float32)
          + bv_ref[...])
    ne_ref[...] = ne
    h = _softplus(ne)
    h = _softplus(
        jnp.dot(h, wo1_ref[...], preferred_element_type=jnp.float32) + bo1_ref[...])
    h = jnp.dot(h, wo2_ref[...], preferred_element_type=jnp.float32) + bo2_ref[...]
    o = ne + h
    pred_ref[...] = (
        jnp.dot(o, wout_ref[...], preferred_element_type=jnp.float32) + bout_ref[...])


def _post_stage(x_i, parts, emb, Wr1, br1, Wr2, br2, gate,
                W_v, b_v, Wo1, bo1, Wo2, bo2, W_out_p, b_out_p):
    n, d = x_i.shape
    nres = Wr1.shape[0]
    row = pl.BlockSpec((_BN, d), lambda i: (i, 0))
    w = pl.BlockSpec((d, d), lambda i: (0, 0))
    b = pl.BlockSpec((1, d), lambda i: (0, 0))
    wr = pl.BlockSpec((nres, d, d), lambda i: (0, 0, 0))
    brs = pl.BlockSpec((nres, 1, d), lambda i: (0, 0, 0))
    pr = pl.BlockSpec((_NC, _BN, d), lambda i: (0, i, 0))
    return pl.pallas_call(
        _post_body,
        grid=(n // _BN,),
        in_specs=[row, pr, row, wr, brs, wr, brs, b, w, b, w, b, w, b, w, b],
        out_specs=[row, row],
        out_shape=[jax.ShapeDtypeStruct((n, d), jnp.float32)] * 2,
    )(x_i, parts, emb, Wr1, br1.reshape(nres, 1, d), Wr2, br2.reshape(nres, 1, d),
      gate.reshape(1, d), W_v, b_v.reshape(1, d), Wo1, bo1.reshape(1, d),
      Wo2, bo2.reshape(1, d), W_out_p, b_out_p)


def kernel(atomic_embedding, f_ij, pair_indices, W_attn, W_i, b_i, W_j, b_j,
           Wr1, br1, Wr2, br2, gate, W_v, b_v, Wo1, bo1, Wo2, bo2, W_out, b_out):
    n, d = atomic_embedding.shape
    e = f_ij.shape[0]
    nprop = W_out.shape[1]
    nw = _NC * _NS
    idx_i3 = pair_indices[0].reshape(nw, e // (nw * _C), _C)
    idx_j3 = pair_indices[1].reshape(nw, e // (nw * _C), _C)

    emb, x_i, t_j = _node_stage(atomic_embedding, W_i, b_i, W_j, b_j)
    g = _attn_stage(f_ij, W_attn)
    parts = _edge_stage(t_j, g, idx_i3, idx_j3)

    W_out_p = jnp.pad(W_out, ((0, 0), (0, d - nprop)))
    b_out_p = jnp.pad(b_out, (0, d - nprop)).reshape(1, d)
    new_emb, pred_p = _post_stage(
        x_i, parts, emb, Wr1, br1, Wr2, br2, gate,
        W_v, b_v, Wo1, bo1, Wo2, bo2, W_out_p, b_out_p)
    return (pred_p[:, :nprop], new_emb)
